# baseline (speedup 1.0000x reference)
import functools

import jax
import jax.numpy as jnp
import numpy as np
from jax import lax
from jax.experimental import pallas as pl
from jax.experimental.pallas import tpu as pltpu
import jax.experimental.pallas.tpu_sc as plsc

N = 100000
E = 1600000
DIM = 64
DEG = 32
NP = 100352
NPK = NP // 8
NC = 2
NS = 16
CHUNK = 100
NROWS = E // CHUNK
RING = 5
RING_MID = 8
ROWS_PER_TILE = NP // NS
ZCHUNK = 64
PBLK = 128

_mesh = plsc.VectorSubcoreMesh(core_axis_name="c", subcore_axis_name="s")
_f32 = jnp.float32
_EYE8 = np.eye(8, dtype=np.float32)
_ROW6 = np.zeros((16, 16), dtype=np.float32)
_ROW6[6, :] = 1.0
_M6 = np.kron(_EYE8, _ROW6)
_HI = jax.lax.Precision.HIGHEST


class _Slot:
    def __init__(self, sbuf, dbuf, rbuf, si, sg, ss):
        self.s, self.d, self.r, self.si, self.sg, self.ss = (
            sbuf, dbuf, rbuf, si, sg, ss)


def _slot_scratch(ring):
    return [
        pltpu.VMEM((ring, CHUNK), jnp.int32),
        pltpu.VMEM((ring, CHUNK), jnp.int32),
        pltpu.VMEM((ring, CHUNK, 16), _f32),
        pltpu.SemaphoreType.DMA,
        pltpu.SemaphoreType.DMA,
        pltpu.SemaphoreType.DMA,
    ]


def _zero_fill(zvm):
    zero16 = jnp.zeros((16,), _f32)
    for i in range(ZCHUNK):
        zvm[i, :] = zero16


def _clear_acc(zvm, acc, r0, sem):
    n = ROWS_PER_TILE // ZCHUNK

    def fire(j, _):
        pltpu.async_copy(zvm, acc.at[pl.ds(r0 + j * ZCHUNK, ZCHUNK), :], sem)
        return 0

    def drain(j, _):
        pltpu.make_async_copy(
            zvm, acc.at[pl.ds(r0, ZCHUNK), :], sem).wait()
        return 0

    lax.fori_loop(0, n, fire, 0)
    lax.fori_loop(0, n, drain, 0)


def _fire_idx(src2, dst2, sl, chunk0, ring):
    pltpu.async_copy(src2.at[pl.ds(chunk0, ring), :], sl.s, sl.si)
    pltpu.async_copy(dst2.at[pl.ds(chunk0, ring), :], sl.d, sl.si)


def _wait_idx(src2, dst2, sl, ring):
    pltpu.make_async_copy(src2.at[pl.ds(0, ring), :], sl.s, sl.si).wait()
    pltpu.make_async_copy(dst2.at[pl.ds(0, ring), :], sl.d, sl.si).wait()


def _fire_gathers(table, sl, ring):
    for k in range(ring):
        pltpu.async_copy(table.at[sl.s.at[k]], sl.r.at[k], sl.sg)


def _wait_gathers(table, sl, ring):
    for k in range(ring):
        pltpu.make_async_copy(table.at[sl.s.at[k]], sl.r.at[k], sl.sg).wait()


def _fire_scatters(acc, sl, ring):
    for k in range(ring):
        pltpu.async_copy(sl.r.at[k], acc.at[sl.d.at[k]], sl.ss, add=True)


def _wait_scatters(acc, sl, ring):
    for k in range(ring):
        pltpu.make_async_copy(sl.r.at[k], acc.at[sl.d.at[k]], sl.ss).wait()


def _sweep(table, src1, dst1, acc, s0, s1, chunk0, n_chunks, ring):
    assert n_chunks % ring == 0
    G = n_chunks // ring
    pairs, rem = divmod(G, 2)
    assert pairs >= 2

    _fire_idx(src1, dst1, s0, chunk0, ring)

    def pair_body(p, _):
        _wait_idx(src1, dst1, s0, ring)
        _fire_gathers(table, s0, ring)

        @pl.when(p >= 1)
        def _():
            _wait_scatters(acc, s1, ring)
        _fire_idx(src1, dst1, s1, chunk0 + (2 * p + 1) * ring, ring)
        _wait_gathers(table, s0, ring)
        _fire_scatters(acc, s0, ring)
        _wait_idx(src1, dst1, s1, ring)
        _fire_gathers(table, s1, ring)
        _wait_scatters(acc, s0, ring)
        if rem:
            _fire_idx(src1, dst1, s0, chunk0 + (2 * p + 2) * ring, ring)
        else:
            @pl.when(p < pairs - 1)
            def _():
                _fire_idx(src1, dst1, s0, chunk0 + (2 * p + 2) * ring, ring)
        _wait_gathers(table, s1, ring)
        _fire_scatters(acc, s1, ring)
        return 0

    lax.fori_loop(0, pairs, pair_body, 0)

    if rem:
        _wait_idx(src1, dst1, s0, ring)
        _fire_gathers(table, s0, ring)
        _wait_scatters(acc, s1, ring)
        _wait_gathers(table, s0, ring)
        _fire_scatters(acc, s0, ring)
        _wait_scatters(acc, s0, ring)
    else:
        _wait_scatters(acc, s1, ring)


def _agg_scratch(ring):
    return (
        _slot_scratch(ring)
        + _slot_scratch(ring)
        + [
            pltpu.VMEM((ZCHUNK, 16), _f32),
            pltpu.VMEM_SHARED((NP, 16), _f32),
            pltpu.SemaphoreType.DMA,
        ]
    )


@functools.partial(
    pl.kernel,
    out_type=jax.ShapeDtypeStruct((2, NP, 16), _f32),
    mesh=_mesh,
    scratch_types=_agg_scratch(RING),
    compiler_params=pltpu.CompilerParams(use_tc_tiling_on_sc=False),
)
def _sc_agg0(x_ref, src_ref, dst_ref, out_ref,
             sb0, db0, rb0, si0, sg0, ss0,
             sb1, db1, rb1, si1, sg1, ss1,
             zvm, acc, zsem):
    c = lax.axis_index("c")
    s = lax.axis_index("s")
    s0 = _Slot(sb0, db0, rb0, si0, sg0, ss0)
    s1 = _Slot(sb1, db1, rb1, si1, sg1, ss1)
    r0 = s * ROWS_PER_TILE
    _zero_fill(zvm)
    _clear_acc(zvm, acc, r0, zsem)
    plsc.subcore_barrier()
    chunks_pt = NROWS // (NC * NS)
    chunk0 = c * (NROWS // NC) + s * chunks_pt
    _sweep(x_ref, src_ref, dst_ref, acc, s0, s1, chunk0, chunks_pt, RING)
    plsc.subcore_barrier()
    pltpu.sync_copy(
        acc.at[pl.ds(r0, ROWS_PER_TILE), :],
        out_ref.at[c, pl.ds(r0, ROWS_PER_TILE), :],
    )


@functools.partial(
    pl.kernel,
    out_type=jax.ShapeDtypeStruct((4, NP, 16), _f32),
    mesh=_mesh,
    scratch_types=_agg_scratch(RING_MID),
    compiler_params=pltpu.CompilerParams(use_tc_tiling_on_sc=False),
)
def _sc_agg(t0, t1, t2, t3, src_ref, dst_ref, out_ref,
            sb0, db0, rb0, si0, sg0, ss0,
            sb1, db1, rb1, si1, sg1, ss1,
            zvm, acc, zsem):
    c = lax.axis_index("c")
    s = lax.axis_index("s")
    s0 = _Slot(sb0, db0, rb0, si0, sg0, ss0)
    s1 = _Slot(sb1, db1, rb1, si1, sg1, ss1)
    r0 = s * ROWS_PER_TILE
    _zero_fill(zvm)
    tables = [t0, t1, t2, t3]
    chunks_pt = NROWS // NS
    chunk0 = s * chunks_pt
    for q in range(4):
        @pl.when(c == q // 2)
        def _(q=q):
            _clear_acc(zvm, acc, r0, zsem)
            plsc.subcore_barrier()
            _sweep(tables[q], src_ref, dst_ref, acc, s0, s1, chunk0,
                   chunks_pt, RING_MID)
            plsc.subcore_barrier()
            pltpu.sync_copy(
                acc.at[pl.ds(r0, ROWS_PER_TILE), :],
                out_ref.at[q, pl.ds(r0, ROWS_PER_TILE), :],
            )
            plsc.subcore_barrier()


@functools.partial(
    pl.kernel,
    out_type=[
        jax.ShapeDtypeStruct((DEG,), _f32),
        jax.ShapeDtypeStruct((16,), _f32),
    ],
    mesh=_mesh,
    scratch_types=[
        pltpu.VMEM((48,), jnp.int32),
        pltpu.VMEM((48, DIM), _f32),
        pltpu.VMEM((1, 128), _f32),
        pltpu.VMEM((1, DIM), _f32),
        pltpu.VMEM((16,), _f32),
        pltpu.VMEM((DEG,), _f32),
        pltpu.VMEM((16,), _f32),
        pltpu.SemaphoreType.DMA,
    ],
    compiler_params=pltpu.CompilerParams(use_tc_tiling_on_sc=False,
                                         needs_layout_passes=False),
)
def _sc_head(h_ref, idx_ref, wlog_ref, wval_ref, bias_ref,
             logit_ref, val_ref,
             ibuf, rbuf, wbuf, vbuf, bbuf, lbuf, obuf, sem):
    c = lax.axis_index("c")
    s = lax.axis_index("s")

    @pl.when((c == 0) & (s == 0))
    def _():
        pltpu.sync_copy(idx_ref, ibuf)
        pltpu.sync_copy(wlog_ref, wbuf)
        pltpu.sync_copy(wval_ref, vbuf)
        pltpu.sync_copy(bias_ref, bbuf)
        pltpu.async_copy(h_ref.at[ibuf], rbuf, sem).wait()
        bb = bbuf[...]
        sa = bb[0]
        sv = bb[1]
        wbv = [wbuf[0, pl.ds(DIM + 16 * k, 16)] for k in range(4)]
        for k in range(4):
            selfk = rbuf[0, pl.ds(16 * k, 16)]
            sa = sa + jnp.sum(selfk * wbuf[0, pl.ds(16 * k, 16)])
            sv = sv + jnp.sum(selfk * vbuf[0, pl.ds(16 * k, 16)])
        for g in range(2):
            rows = jax.lax.iota(jnp.int32, 16) + (16 + 16 * g)
            acc = jnp.full((16,), sa, _f32)
            for j in range(DIM):
                col = plsc.load_gather(rbuf,
                                       [rows, jnp.full((16,), j, jnp.int32)])
                acc = acc + col * wbv[j // 16][j % 16]
            lbuf[pl.ds(16 * g, 16)] = acc
        obuf[...] = jnp.full((16,), sv, _f32)
        pltpu.sync_copy(lbuf, logit_ref)
        pltpu.sync_copy(obuf, val_ref)


def _tc_layer0_body(bdl_ref, bdr_ref, bias_ref, m_ref, part_ref, x_ref,
                    q0_ref, q1_ref, q2_ref, q3_ref, cinv_ref):
    x_sum = part_ref[0] + part_ref[1]
    cnt = jnp.dot(x_sum, m_ref[...], precision=_HI)
    cinv = 1.0 / jnp.maximum(cnt, 1.0)
    mean = x_sum * cinv
    xp = x_ref[...]
    outs = (q0_ref, q1_ref, q2_ref, q3_ref)
    for q in range(4):
        h = (mean @ bdl_ref[q] + xp @ bdr_ref[q]) + bias_ref[q][None, :]
        outs[q][...] = jnp.maximum(h, 0.0)
    cinv_ref[...] = cinv


def _tc_layer_mid_body(bdl_ref, bdr_ref, bias_ref, agg_ref, cinv_ref,
                       h0_ref, h1_ref, h2_ref, h3_ref,
                       q0_ref, q1_ref, q2_ref, q3_ref):
    cinv = cinv_ref[...]
    means = [agg_ref[qi] * cinv for qi in range(4)]
    hs = [h0_ref[...], h1_ref[...], h2_ref[...], h3_ref[...]]
    outs = (q0_ref, q1_ref, q2_ref, q3_ref)
    for qo in range(4):
        acc = bias_ref[qo][None, :]
        for qi in range(4):
            acc = acc + means[qi] @ bdl_ref[qi, qo] + hs[qi] @ bdr_ref[qi, qo]
        outs[qo][...] = jnp.maximum(acc, 0.0)


def _tc_layer_last_body(bdl_ref, bdr_ref, bias_ref, agg_ref, cinv_ref,
                        h0_ref, h1_ref, h2_ref, h3_ref, out_ref):
    cinv = cinv_ref[...]
    hs = [h0_ref[...], h1_ref[...], h2_ref[...], h3_ref[...]]
    acc = bias_ref[...]
    for qi in range(4):
        mean = agg_ref[qi] * cinv
        acc = acc + mean @ bdl_ref[qi] + hs[qi] @ bdr_ref[qi]
    out_ref[...] = acc


_PGRID = NPK // PBLK


def _spec(shape, imap):
    return pl.BlockSpec(shape, imap)


_p_spec = _spec((PBLK, 128), lambda i: (i, 0))
_bd44 = _spec((4, 4, 128, 128), lambda i: (0, 0, 0, 0))
_bd4 = _spec((4, 128, 128), lambda i: (0, 0, 0))
_bias4 = _spec((4, 128), lambda i: (0, 0))

_tc_layer0 = pl.pallas_call(
    _tc_layer0_body,
    grid=(_PGRID,),
    in_specs=[
        _bd4, _bd4, _bias4,
        _spec((128, 128), lambda i: (0, 0)),
        _spec((2, PBLK, 128), lambda i: (0, i, 0)),
        _p_spec,
    ],
    out_specs=[_p_spec] * 5,
    out_shape=[jax.ShapeDtypeStruct((NPK, 128), _f32)] * 5,
)

_tc_layer_mid = pl.pallas_call(
    _tc_layer_mid_body,
    grid=(_PGRID,),
    in_specs=[
        _bd44, _bd44, _bias4,
        _spec((4, PBLK, 128), lambda i: (0, i, 0)),
        _p_spec, _p_spec, _p_spec, _p_spec, _p_spec,
    ],
    out_specs=[_p_spec] * 4,
    out_shape=[jax.ShapeDtypeStruct((NPK, 128), _f32)] * 4,
)

_tc_layer_last = pl.pallas_call(
    _tc_layer_last_body,
    grid=(_PGRID,),
    in_specs=[
        _spec((4, 128, 512), lambda i: (0, 0, 0)),
        _spec((4, 128, 512), lambda i: (0, 0, 0)),
        _spec((1, 512), lambda i: (0, 0)),
        _spec((4, PBLK, 128), lambda i: (0, i, 0)),
        _p_spec, _p_spec, _p_spec, _p_spec, _p_spec,
    ],
    out_specs=[_spec((PBLK, 512), lambda i: (i, 0))],
    out_shape=[jax.ShapeDtypeStruct((NPK, 512), _f32)],
)

_E8A = _EYE8.reshape(1, 1, 8, 1, 8, 1)
_E8B = _EYE8.reshape(1, 8, 1, 8, 1)


def _bd_mid(wt):
    p = wt.reshape(4, 16, 4, 16).transpose(0, 2, 1, 3)
    return (_E8A * p[:, :, None, :, None, :]).reshape(4, 4, 128, 128)


def _bd_in(wt16):
    p = wt16.reshape(16, 4, 16).transpose(1, 0, 2)
    return (_E8B * p[:, None, :, None, :]).reshape(4, 128, 128)


def _bd_last(wt):
    p = wt.reshape(4, 16, 64)
    return (_E8B * p[:, None, :, None, :]).reshape(4, 128, 512)


def kernel(has_evader, has_pursuer, goal_distance, decoy_distance,
           remaining_time, deceptiveness, edge_index, self_node,
           Wl0, bl0, Wr0, Wl1, bl1, Wr1, Wl2, bl2, Wr2, Wl3, bl3, Wr3,
           W_logit, b_logit, W_value, b_value):
    feats = jnp.stack([has_evader, has_pursuer, goal_distance, decoy_distance,
                       remaining_time, deceptiveness], axis=-1).astype(_f32)
    f3 = jnp.concatenate([feats, jnp.zeros((NP - N, 6), _f32)],
                         axis=0).reshape(NPK, 8, 6)
    xpacked = jnp.concatenate(
        [f3, jnp.ones((NPK, 8, 1), _f32), jnp.zeros((NPK, 8, 9), _f32)],
        axis=2).reshape(NPK, 128)
    x16 = xpacked.reshape(NP, 16)

    src2 = edge_index[0].astype(jnp.int32).reshape(NROWS, CHUNK)
    dst2 = edge_index[1].astype(jnp.int32).reshape(NROWS, CHUNK)

    wl0 = jnp.zeros((16, DIM), _f32).at[:6, :].set(Wl0.T)
    wr0 = jnp.zeros((16, DIM), _f32).at[:6, :].set(Wr0.T)

    part = _sc_agg0(x16, src2, dst2)
    partp = part.reshape(2, NPK, 128)
    bias0 = jnp.tile(bl0.reshape(4, 16), (1, 8))
    h0 = _tc_layer0(_bd_in(wl0), _bd_in(wr0), bias0, _M6, partp, xpacked)
    q, cinv = h0[:4], h0[4]

    for li, (Wl, bl, Wr) in enumerate(
            ((Wl1, bl1, Wr1), (Wl2, bl2, Wr2), (Wl3, bl3, Wr3))):
        last = li == 2
        t = [qq.reshape(NP, 16) for qq in q]
        agg = _sc_agg(t[0], t[1], t[2], t[3], src2, dst2)
        aggp = agg.reshape(4, NPK, 128)
        if last:
            h3p = _tc_layer_last(_bd_last(Wl.T), _bd_last(Wr.T),
                                 jnp.tile(bl, 8).reshape(1, 512),
                                 aggp, cinv, q[0], q[1], q[2], q[3])[0]
        else:
            bias = jnp.tile(bl.reshape(4, 16), (1, 8))
            q = _tc_layer_mid(_bd_mid(Wl.T), _bd_mid(Wr.T), bias,
                              aggp, cinv, q[0], q[1], q[2], q[3])

    h3 = h3p.reshape(NP, DIM)
    idx = jnp.concatenate([
        jnp.full((16,), self_node, jnp.int32),
        edge_index[1, :DEG].astype(jnp.int32),
    ]).astype(jnp.int32)
    bias_lv = jnp.concatenate(
        [b_logit, b_value, jnp.zeros((14,), _f32)]).astype(_f32)
    logits, value16 = _sc_head(h3, idx, W_logit, W_value, bias_lv)
    return logits, value16[0]


# speedup vs baseline: 1.0043x; 1.0043x over previous
import functools

import jax
import jax.numpy as jnp
import numpy as np
from jax import lax
from jax.experimental import pallas as pl
from jax.experimental.pallas import tpu as pltpu
import jax.experimental.pallas.tpu_sc as plsc

N = 100000
E = 1600000
DIM = 64
DEG = 32
NP = 100352
NPK = NP // 8
NC = 2
NS = 16
CHUNK = 80
RING = 5
RING_MID = 10
ROWS_PER_TILE = NP // NS
ZCHUNK = 64
PBLK = 128

_mesh = plsc.VectorSubcoreMesh(core_axis_name="c", subcore_axis_name="s")
_f32 = jnp.float32
_EYE8 = np.eye(8, dtype=np.float32)
_ROW6 = np.zeros((16, 16), dtype=np.float32)
_ROW6[6, :] = 1.0
_M6 = np.kron(_EYE8, _ROW6)
_HI = jax.lax.Precision.HIGHEST


class _Slot:
    def __init__(self, sbuf, dbuf, rbuf, si, sg, ss):
        self.s, self.d, self.r, self.si, self.sg, self.ss = (
            sbuf, dbuf, rbuf, si, sg, ss)


def _slot_scratch(ring):
    return [
        pltpu.VMEM((ring, CHUNK), jnp.int32),
        pltpu.VMEM((ring, CHUNK), jnp.int32),
        pltpu.VMEM((ring, CHUNK, 16), _f32),
        pltpu.SemaphoreType.DMA,
        pltpu.SemaphoreType.DMA,
        pltpu.SemaphoreType.DMA,
    ]


def _zero_fill(zvm):
    zero16 = jnp.zeros((16,), _f32)
    for i in range(ZCHUNK):
        zvm[i, :] = zero16


def _clear_acc(zvm, acc, r0, sem):
    n = ROWS_PER_TILE // ZCHUNK

    def fire(j, _):
        pltpu.async_copy(zvm, acc.at[pl.ds(r0 + j * ZCHUNK, ZCHUNK), :], sem)
        return 0

    def drain(j, _):
        pltpu.make_async_copy(
            zvm, acc.at[pl.ds(r0, ZCHUNK), :], sem).wait()
        return 0

    lax.fori_loop(0, n, fire, 0)
    lax.fori_loop(0, n, drain, 0)


def _fire_idx(src1, dst1, sl, chunk0, ring):
    base = pl.multiple_of(chunk0 * CHUNK, 8)
    for k in range(ring):
        pltpu.async_copy(src1.at[pl.ds(base + k * CHUNK, CHUNK)],
                         sl.s.at[k], sl.si)
        pltpu.async_copy(dst1.at[pl.ds(base + k * CHUNK, CHUNK)],
                         sl.d.at[k], sl.si)


def _wait_idx(src1, dst1, sl, ring):
    for k in range(ring):
        pltpu.make_async_copy(src1.at[pl.ds(0, CHUNK)], sl.s.at[k],
                              sl.si).wait()
        pltpu.make_async_copy(dst1.at[pl.ds(0, CHUNK)], sl.d.at[k],
                              sl.si).wait()


def _fire_gathers(table, sl, ring):
    for k in range(ring):
        pltpu.async_copy(table.at[sl.s.at[k]], sl.r.at[k], sl.sg)


def _wait_gathers(table, sl, ring):
    for k in range(ring):
        pltpu.make_async_copy(table.at[sl.s.at[k]], sl.r.at[k], sl.sg).wait()


def _fire_scatters(acc, sl, ring):
    for k in range(ring):
        pltpu.async_copy(sl.r.at[k], acc.at[sl.d.at[k]], sl.ss, add=True)


def _wait_scatters(acc, sl, ring):
    for k in range(ring):
        pltpu.make_async_copy(sl.r.at[k], acc.at[sl.d.at[k]], sl.ss).wait()


def _sweep(table, src1, dst1, acc, s0, s1, chunk0, n_chunks, ring):
    assert n_chunks % ring == 0
    G = n_chunks // ring
    pairs, rem = divmod(G, 2)
    assert pairs >= 2

    _fire_idx(src1, dst1, s0, chunk0, ring)

    def pair_body(p, _):
        _wait_idx(src1, dst1, s0, ring)
        _fire_gathers(table, s0, ring)

        @pl.when(p >= 1)
        def _():
            _wait_scatters(acc, s1, ring)
        _fire_idx(src1, dst1, s1, chunk0 + (2 * p + 1) * ring, ring)
        _wait_gathers(table, s0, ring)
        _fire_scatters(acc, s0, ring)
        _wait_idx(src1, dst1, s1, ring)
        _fire_gathers(table, s1, ring)
        _wait_scatters(acc, s0, ring)
        if rem:
            _fire_idx(src1, dst1, s0, chunk0 + (2 * p + 2) * ring, ring)
        else:
            @pl.when(p < pairs - 1)
            def _():
                _fire_idx(src1, dst1, s0, chunk0 + (2 * p + 2) * ring, ring)
        _wait_gathers(table, s1, ring)
        _fire_scatters(acc, s1, ring)
        return 0

    lax.fori_loop(0, pairs, pair_body, 0)

    if rem:
        _wait_idx(src1, dst1, s0, ring)
        _fire_gathers(table, s0, ring)
        _wait_scatters(acc, s1, ring)
        _wait_gathers(table, s0, ring)
        _fire_scatters(acc, s0, ring)
        _wait_scatters(acc, s0, ring)
    else:
        _wait_scatters(acc, s1, ring)


def _agg_scratch(ring):
    return (
        _slot_scratch(ring)
        + _slot_scratch(ring)
        + [
            pltpu.VMEM((ZCHUNK, 16), _f32),
            pltpu.VMEM_SHARED((NP, 16), _f32),
            pltpu.SemaphoreType.DMA,
        ]
    )


@functools.partial(
    pl.kernel,
    out_type=jax.ShapeDtypeStruct((2, NP, 16), _f32),
    mesh=_mesh,
    scratch_types=_agg_scratch(RING),
    compiler_params=pltpu.CompilerParams(use_tc_tiling_on_sc=False),
)
def _sc_agg0(x_ref, src_ref, dst_ref, out_ref,
             sb0, db0, rb0, si0, sg0, ss0,
             sb1, db1, rb1, si1, sg1, ss1,
             zvm, acc, zsem):
    c = lax.axis_index("c")
    s = lax.axis_index("s")
    s0 = _Slot(sb0, db0, rb0, si0, sg0, ss0)
    s1 = _Slot(sb1, db1, rb1, si1, sg1, ss1)
    r0 = s * ROWS_PER_TILE
    _zero_fill(zvm)
    _clear_acc(zvm, acc, r0, zsem)
    plsc.subcore_barrier()
    nrows = E // CHUNK
    chunks_pt = nrows // (NC * NS)
    chunk0 = c * (nrows // NC) + s * chunks_pt
    _sweep(x_ref, src_ref, dst_ref, acc, s0, s1, chunk0, chunks_pt, RING)
    plsc.subcore_barrier()
    pltpu.sync_copy(
        acc.at[pl.ds(r0, ROWS_PER_TILE), :],
        out_ref.at[c, pl.ds(r0, ROWS_PER_TILE), :],
    )


@functools.partial(
    pl.kernel,
    out_type=jax.ShapeDtypeStruct((4, NP, 16), _f32),
    mesh=_mesh,
    scratch_types=_agg_scratch(RING_MID),
    compiler_params=pltpu.CompilerParams(use_tc_tiling_on_sc=False),
)
def _sc_agg(t0, t1, t2, t3, src_ref, dst_ref, out_ref,
            sb0, db0, rb0, si0, sg0, ss0,
            sb1, db1, rb1, si1, sg1, ss1,
            zvm, acc, zsem):
    c = lax.axis_index("c")
    s = lax.axis_index("s")
    s0 = _Slot(sb0, db0, rb0, si0, sg0, ss0)
    s1 = _Slot(sb1, db1, rb1, si1, sg1, ss1)
    r0 = s * ROWS_PER_TILE
    _zero_fill(zvm)
    tables = [t0, t1, t2, t3]
    chunks_pt = (E // CHUNK) // NS
    chunk0 = s * chunks_pt
    for q in range(4):
        @pl.when(c == q // 2)
        def _(q=q):
            _clear_acc(zvm, acc, r0, zsem)
            plsc.subcore_barrier()
            _sweep(tables[q], src_ref, dst_ref, acc, s0, s1, chunk0,
                   chunks_pt, RING_MID)
            plsc.subcore_barrier()
            pltpu.sync_copy(
                acc.at[pl.ds(r0, ROWS_PER_TILE), :],
                out_ref.at[q, pl.ds(r0, ROWS_PER_TILE), :],
            )
            plsc.subcore_barrier()


@functools.partial(
    pl.kernel,
    out_type=[
        jax.ShapeDtypeStruct((DEG,), _f32),
        jax.ShapeDtypeStruct((16,), _f32),
    ],
    mesh=_mesh,
    scratch_types=[
        pltpu.VMEM((48,), jnp.int32),
        pltpu.VMEM((48, DIM), _f32),
        pltpu.VMEM((1, 128), _f32),
        pltpu.VMEM((1, DIM), _f32),
        pltpu.VMEM((16,), _f32),
        pltpu.VMEM((DEG,), _f32),
        pltpu.VMEM((16,), _f32),
        pltpu.SemaphoreType.DMA,
    ],
    compiler_params=pltpu.CompilerParams(use_tc_tiling_on_sc=False,
                                         needs_layout_passes=False),
)
def _sc_head(h_ref, idx_ref, wlog_ref, wval_ref, bias_ref,
             logit_ref, val_ref,
             ibuf, rbuf, wbuf, vbuf, bbuf, lbuf, obuf, sem):
    c = lax.axis_index("c")
    s = lax.axis_index("s")

    @pl.when((c == 0) & (s == 0))
    def _():
        pltpu.sync_copy(idx_ref, ibuf)
        pltpu.sync_copy(wlog_ref, wbuf)
        pltpu.sync_copy(wval_ref, vbuf)
        pltpu.sync_copy(bias_ref, bbuf)
        pltpu.async_copy(h_ref.at[ibuf], rbuf, sem).wait()
        bb = bbuf[...]
        sa = bb[0]
        sv = bb[1]
        wbv = [wbuf[0, pl.ds(DIM + 16 * k, 16)] for k in range(4)]
        for k in range(4):
            selfk = rbuf[0, pl.ds(16 * k, 16)]
            sa = sa + jnp.sum(selfk * wbuf[0, pl.ds(16 * k, 16)])
            sv = sv + jnp.sum(selfk * vbuf[0, pl.ds(16 * k, 16)])
        for g in range(2):
            rows = jax.lax.iota(jnp.int32, 16) + (16 + 16 * g)
            acc = jnp.full((16,), sa, _f32)
            for j in range(DIM):
                col = plsc.load_gather(rbuf,
                                       [rows, jnp.full((16,), j, jnp.int32)])
                acc = acc + col * wbv[j // 16][j % 16]
            lbuf[pl.ds(16 * g, 16)] = acc
        obuf[...] = jnp.full((16,), sv, _f32)
        pltpu.sync_copy(lbuf, logit_ref)
        pltpu.sync_copy(obuf, val_ref)


def _tc_layer0_body(bdl_ref, bdr_ref, bias_ref, m_ref, part_ref, x_ref,
                    q0_ref, q1_ref, q2_ref, q3_ref, cinv_ref):
    x_sum = part_ref[0] + part_ref[1]
    cnt = jnp.dot(x_sum, m_ref[...], precision=_HI)
    cinv = 1.0 / jnp.maximum(cnt, 1.0)
    mean = x_sum * cinv
    xp = x_ref[...]
    outs = (q0_ref, q1_ref, q2_ref, q3_ref)
    for q in range(4):
        h = (mean @ bdl_ref[q] + xp @ bdr_ref[q]) + bias_ref[q][None, :]
        outs[q][...] = jnp.maximum(h, 0.0)
    cinv_ref[...] = cinv


def _tc_layer_mid_body(bdl_ref, bdr_ref, bias_ref, agg_ref, cinv_ref,
                       h0_ref, h1_ref, h2_ref, h3_ref,
                       q0_ref, q1_ref, q2_ref, q3_ref):
    cinv = cinv_ref[...]
    means = [agg_ref[qi] * cinv for qi in range(4)]
    hs = [h0_ref[...], h1_ref[...], h2_ref[...], h3_ref[...]]
    outs = (q0_ref, q1_ref, q2_ref, q3_ref)
    for qo in range(4):
        acc = bias_ref[qo][None, :]
        for qi in range(4):
            acc = acc + means[qi] @ bdl_ref[qi, qo] + hs[qi] @ bdr_ref[qi, qo]
        outs[qo][...] = jnp.maximum(acc, 0.0)


def _tc_layer_last_body(bdl_ref, bdr_ref, bias_ref, agg_ref, cinv_ref,
                        h0_ref, h1_ref, h2_ref, h3_ref, out_ref):
    cinv = cinv_ref[...]
    hs = [h0_ref[...], h1_ref[...], h2_ref[...], h3_ref[...]]
    acc = bias_ref[...]
    for qi in range(4):
        mean = agg_ref[qi] * cinv
        acc = acc + mean @ bdl_ref[qi] + hs[qi] @ bdr_ref[qi]
    out_ref[...] = acc


_PGRID = NPK // PBLK


def _spec(shape, imap):
    return pl.BlockSpec(shape, imap)


_p_spec = _spec((PBLK, 128), lambda i: (i, 0))
_bd44 = _spec((4, 4, 128, 128), lambda i: (0, 0, 0, 0))
_bd4 = _spec((4, 128, 128), lambda i: (0, 0, 0))
_bias4 = _spec((4, 128), lambda i: (0, 0))

_tc_layer0 = pl.pallas_call(
    _tc_layer0_body,
    grid=(_PGRID,),
    in_specs=[
        _bd4, _bd4, _bias4,
        _spec((128, 128), lambda i: (0, 0)),
        _spec((2, PBLK, 128), lambda i: (0, i, 0)),
        _p_spec,
    ],
    out_specs=[_p_spec] * 5,
    out_shape=[jax.ShapeDtypeStruct((NPK, 128), _f32)] * 5,
)

_tc_layer_mid = pl.pallas_call(
    _tc_layer_mid_body,
    grid=(_PGRID,),
    in_specs=[
        _bd44, _bd44, _bias4,
        _spec((4, PBLK, 128), lambda i: (0, i, 0)),
        _p_spec, _p_spec, _p_spec, _p_spec, _p_spec,
    ],
    out_specs=[_p_spec] * 4,
    out_shape=[jax.ShapeDtypeStruct((NPK, 128), _f32)] * 4,
)

_tc_layer_last = pl.pallas_call(
    _tc_layer_last_body,
    grid=(_PGRID,),
    in_specs=[
        _spec((4, 128, 512), lambda i: (0, 0, 0)),
        _spec((4, 128, 512), lambda i: (0, 0, 0)),
        _spec((1, 512), lambda i: (0, 0)),
        _spec((4, PBLK, 128), lambda i: (0, i, 0)),
        _p_spec, _p_spec, _p_spec, _p_spec, _p_spec,
    ],
    out_specs=[_spec((PBLK, 512), lambda i: (i, 0))],
    out_shape=[jax.ShapeDtypeStruct((NPK, 512), _f32)],
)

_E8A = _EYE8.reshape(1, 1, 8, 1, 8, 1)
_E8B = _EYE8.reshape(1, 8, 1, 8, 1)


def _bd_mid(wt):
    p = wt.reshape(4, 16, 4, 16).transpose(0, 2, 1, 3)
    return (_E8A * p[:, :, None, :, None, :]).reshape(4, 4, 128, 128)


def _bd_in(wt16):
    p = wt16.reshape(16, 4, 16).transpose(1, 0, 2)
    return (_E8B * p[:, None, :, None, :]).reshape(4, 128, 128)


def _bd_last(wt):
    p = wt.reshape(4, 16, 64)
    return (_E8B * p[:, None, :, None, :]).reshape(4, 128, 512)


def kernel(has_evader, has_pursuer, goal_distance, decoy_distance,
           remaining_time, deceptiveness, edge_index, self_node,
           Wl0, bl0, Wr0, Wl1, bl1, Wr1, Wl2, bl2, Wr2, Wl3, bl3, Wr3,
           W_logit, b_logit, W_value, b_value):
    feats = jnp.stack([has_evader, has_pursuer, goal_distance, decoy_distance,
                       remaining_time, deceptiveness], axis=-1).astype(_f32)
    f3 = jnp.concatenate([feats, jnp.zeros((NP - N, 6), _f32)],
                         axis=0).reshape(NPK, 8, 6)
    xpacked = jnp.concatenate(
        [f3, jnp.ones((NPK, 8, 1), _f32), jnp.zeros((NPK, 8, 9), _f32)],
        axis=2).reshape(NPK, 128)
    x16 = xpacked.reshape(NP, 16)

    src1 = edge_index[0].astype(jnp.int32)
    dst1 = edge_index[1].astype(jnp.int32)

    wl0 = jnp.zeros((16, DIM), _f32).at[:6, :].set(Wl0.T)
    wr0 = jnp.zeros((16, DIM), _f32).at[:6, :].set(Wr0.T)

    part = _sc_agg0(x16, src1, dst1)
    partp = part.reshape(2, NPK, 128)
    bias0 = jnp.tile(bl0.reshape(4, 16), (1, 8))
    h0 = _tc_layer0(_bd_in(wl0), _bd_in(wr0), bias0, _M6, partp, xpacked)
    q, cinv = h0[:4], h0[4]

    for li, (Wl, bl, Wr) in enumerate(
            ((Wl1, bl1, Wr1), (Wl2, bl2, Wr2), (Wl3, bl3, Wr3))):
        last = li == 2
        t = [qq.reshape(NP, 16) for qq in q]
        agg = _sc_agg(t[0], t[1], t[2], t[3], src1, dst1)
        aggp = agg.reshape(4, NPK, 128)
        if last:
            h3p = _tc_layer_last(_bd_last(Wl.T), _bd_last(Wr.T),
                                 jnp.tile(bl, 8).reshape(1, 512),
                                 aggp, cinv, q[0], q[1], q[2], q[3])[0]
        else:
            bias = jnp.tile(bl.reshape(4, 16), (1, 8))
            q = _tc_layer_mid(_bd_mid(Wl.T), _bd_mid(Wr.T), bias,
                              aggp, cinv, q[0], q[1], q[2], q[3])

    h3 = h3p.reshape(NP, DIM)
    idx = jnp.concatenate([
        jnp.full((16,), self_node, jnp.int32),
        edge_index[1, :DEG].astype(jnp.int32),
    ]).astype(jnp.int32)
    bias_lv = jnp.concatenate(
        [b_logit, b_value, jnp.zeros((14,), _f32)]).astype(_f32)
    logits, value16 = _sc_head(h3, idx, W_logit, W_value, bias_lv)
    return logits, value16[0]
